# baseline (speedup 1.0000x reference)
import functools

import jax
import jax.numpy as jnp
from jax import lax
from jax.experimental import pallas as pl
from jax.experimental.pallas import tpu as pltpu
from jax.experimental.pallas import tpu_sc as plsc

N = 10000
E = N * 32
H = 128

NC = 2
NS = 16
NW = NC * NS
EPW = E // NW
C = 100
CHUNKS = EPW // C
NBUF = 3
ROWS_PER_TILE = 624
TAIL_ROWS = N - NS * ROWS_PER_TILE

_NEG_SLOPE = 0.2


def _lrelu(x):
    return jnp.where(x >= 0, x, _NEG_SLOPE * x)


def _make_sc_agg(compute_deg: bool):
    mesh = plsc.VectorSubcoreMesh(core_axis_name="c", subcore_axis_name="s")
    if compute_deg:
        out_type = [jax.ShapeDtypeStruct((NC, N, H), jnp.float32),
                    jax.ShapeDtypeStruct((NC, N), jnp.float32)]
    else:
        out_type = jax.ShapeDtypeStruct((NC, N, H), jnp.float32)
    scratch_types = (
        [pltpu.VMEM((2, C), jnp.int32) for _ in range(NBUF)]
        + [pltpu.VMEM((C, H), jnp.float32) for _ in range(NBUF)]
        + [
            pltpu.VMEM((128,), jnp.float32),
            pltpu.VMEM_SHARED((N, H), jnp.float32),
            pltpu.VMEM_SHARED((N,), jnp.float32),
        ]
        + [pltpu.SemaphoreType.DMA for _ in range(NBUF)]
    )

    def body(h_hbm, idx_hbm, zrows_hbm, zdeg_hbm, *refs):
        if compute_deg:
            agg_out, deg_out = refs[0], refs[1]
            rest = refs[2:]
        else:
            agg_out = refs[0]
            deg_out = None
            rest = refs[1:]
        idxb = rest[:NBUF]
        rowsb = rest[NBUF:2 * NBUF]
        ones_v, agg_sh, deg_sh = rest[2 * NBUF:2 * NBUF + 3]
        semb = rest[2 * NBUF + 3:]

        c = lax.axis_index("c")
        s = lax.axis_index("s")
        wid = c * NS + s

        pltpu.sync_copy(zrows_hbm,
                        agg_sh.at[pl.ds(s * ROWS_PER_TILE, ROWS_PER_TILE)])

        @pl.when(s == NS - 1)
        def _():
            pltpu.sync_copy(zrows_hbm.at[pl.ds(0, TAIL_ROWS)],
                            agg_sh.at[pl.ds(NS * ROWS_PER_TILE, TAIL_ROWS)])

        if compute_deg:
            @pl.when(s == 0)
            def _():
                pltpu.sync_copy(zdeg_hbm, deg_sh)
            one16 = jnp.ones((16,), jnp.float32)
            for j in range(8):
                ones_v[pl.ds(j * 16, 16)] = one16
        plsc.subcore_barrier()

        def scatter_chunk(idx_v, rows_v):
            pltpu.sync_copy(rows_v, agg_sh.at[idx_v.at[1]], add=True)
            if compute_deg:
                pltpu.sync_copy(ones_v.at[pl.ds(0, C)],
                                deg_sh.at[idx_v.at[1]], add=True)

        for b in range(NBUF):
            pltpu.sync_copy(idx_hbm.at[wid, b], idxb[b])
            pltpu.async_copy(h_hbm.at[idxb[b].at[0]], rowsb[b], semb[b])

        def round_body(i, carry):
            k0 = i * NBUF
            for b in range(NBUF):
                pltpu.make_async_copy(h_hbm.at[idxb[b].at[0]],
                                      rowsb[b], semb[b]).wait()
                scatter_chunk(idxb[b], rowsb[b])

                @pl.when(k0 + b + NBUF < CHUNKS)
                def _(b=b, k=k0 + b):
                    pltpu.sync_copy(idx_hbm.at[wid, k + NBUF], idxb[b])
                    pltpu.async_copy(h_hbm.at[idxb[b].at[0]], rowsb[b], semb[b])
            return carry

        lax.fori_loop(0, CHUNKS // NBUF, round_body, 0)
        for k in range((CHUNKS // NBUF) * NBUF, CHUNKS):
            b = k % NBUF
            pltpu.make_async_copy(h_hbm.at[idxb[b].at[0]],
                                  rowsb[b], semb[b]).wait()
            scatter_chunk(idxb[b], rowsb[b])
        plsc.subcore_barrier()

        r0 = s * ROWS_PER_TILE
        pltpu.sync_copy(agg_sh.at[pl.ds(r0, ROWS_PER_TILE)],
                        agg_out.at[c, pl.ds(r0, ROWS_PER_TILE)])

        @pl.when(s == NS - 1)
        def _():
            rt = NS * ROWS_PER_TILE
            pltpu.sync_copy(agg_sh.at[pl.ds(rt, TAIL_ROWS)],
                            agg_out.at[c, pl.ds(rt, TAIL_ROWS)])

        if compute_deg:
            @pl.when(s == 0)
            def _():
                pltpu.sync_copy(deg_sh, deg_out.at[c])

    return functools.partial(pl.kernel, mesh=mesh, out_type=out_type,
                             scratch_types=scratch_types)(body)


_sc_agg_deg = _make_sc_agg(True)
_sc_agg = _make_sc_agg(False)


BLK = 1000


def _embed_body(x_ref, w_ref, b_ref, o_ref):
    o_ref[...] = jnp.dot(x_ref[...], w_ref[...],
                         preferred_element_type=jnp.float32) + b_ref[...]


def _tc_embed(x, w, b):
    d = x.shape[1]
    return pl.pallas_call(
        _embed_body,
        grid=(N // BLK,),
        in_specs=[
            pl.BlockSpec((BLK, d), lambda i: (i, 0)),
            pl.BlockSpec((d, H), lambda i: (0, 0)),
            pl.BlockSpec((1, H), lambda i: (0, 0)),
        ],
        out_specs=pl.BlockSpec((BLK, H), lambda i: (i, 0)),
        out_shape=jax.ShapeDtypeStruct((N, H), jnp.float32),
    )(x, w, b.reshape(1, H))


def _layer_body(p_ref, deg_ref, w1_ref, b1_ref, w2_ref, b2_ref, o_ref):
    agg = p_ref[0] + p_ref[1]
    deg = deg_ref[0] + deg_ref[1]
    agg = agg / jnp.maximum(deg, 1.0)
    t = _lrelu(jnp.dot(agg, w1_ref[...],
                       preferred_element_type=jnp.float32) + b1_ref[...])
    o_ref[...] = _lrelu(jnp.dot(t, w2_ref[...],
                                preferred_element_type=jnp.float32) + b2_ref[...])


def _tc_layer(partials, degp, w1, b1, w2, b2):
    return pl.pallas_call(
        _layer_body,
        grid=(N // BLK,),
        in_specs=[
            pl.BlockSpec((NC, BLK, H), lambda i: (0, i, 0)),
            pl.BlockSpec((NC, BLK, 1), lambda i: (0, i, 0)),
            pl.BlockSpec((H, H), lambda i: (0, 0)),
            pl.BlockSpec((1, H), lambda i: (0, 0)),
            pl.BlockSpec((H, H), lambda i: (0, 0)),
            pl.BlockSpec((1, H), lambda i: (0, 0)),
        ],
        out_specs=pl.BlockSpec((BLK, H), lambda i: (i, 0)),
        out_shape=jax.ShapeDtypeStruct((N, H), jnp.float32),
    )(partials, degp, w1, b1.reshape(1, H), w2, b2.reshape(1, H))


def _layer_ro_body(p_ref, deg_ref, w1_ref, b1_ref, w2_ref, b2_ref,
                   wro_ref, bro_ref, o_ref):
    agg = p_ref[0] + p_ref[1]
    deg = deg_ref[0] + deg_ref[1]
    agg = agg / jnp.maximum(deg, 1.0)
    t = _lrelu(jnp.dot(agg, w1_ref[...],
                       preferred_element_type=jnp.float32) + b1_ref[...])
    h = _lrelu(jnp.dot(t, w2_ref[...],
                       preferred_element_type=jnp.float32) + b2_ref[...])
    o_ref[...] = jnp.dot(h, wro_ref[...],
                         preferred_element_type=jnp.float32) + bro_ref[...]


def _tc_layer_ro(partials, degp, w1, b1, w2, b2, wro, bro):
    o = wro.shape[1]
    return pl.pallas_call(
        _layer_ro_body,
        grid=(N // BLK,),
        in_specs=[
            pl.BlockSpec((NC, BLK, H), lambda i: (0, i, 0)),
            pl.BlockSpec((NC, BLK, 1), lambda i: (0, i, 0)),
            pl.BlockSpec((H, H), lambda i: (0, 0)),
            pl.BlockSpec((1, H), lambda i: (0, 0)),
            pl.BlockSpec((H, H), lambda i: (0, 0)),
            pl.BlockSpec((1, H), lambda i: (0, 0)),
            pl.BlockSpec((H, o), lambda i: (0, 0)),
            pl.BlockSpec((1, o), lambda i: (0, 0)),
        ],
        out_specs=pl.BlockSpec((BLK, o), lambda i: (i, 0)),
        out_shape=jax.ShapeDtypeStruct((N, o), jnp.float32),
    )(partials, degp, w1, b1.reshape(1, H), w2, b2.reshape(1, H),
      wro, bro.reshape(1, o))


def kernel(feats, edge_index, W_emb, b_emb, W1_0, b1_0, W2_0, b2_0,
           W1_1, b1_1, W2_1, b2_1, W_ro, b_ro):
    idx = jnp.stack([edge_index[0].reshape(NW, CHUNKS, C),
                     edge_index[1].reshape(NW, CHUNKS, C)], axis=2)
    zrows = jnp.zeros((ROWS_PER_TILE, H), jnp.float32)
    zdeg = jnp.zeros((N,), jnp.float32)

    h0 = _tc_embed(feats, W_emb, b_emb)
    aggp, degp = _sc_agg_deg(h0, idx, zrows, zdeg)
    degp3 = degp.reshape(NC, N, 1)
    h1 = _tc_layer(aggp, degp3, W1_0, b1_0, W2_0, b2_0)
    aggp2 = _sc_agg(h1, idx, zrows, zdeg)
    return _tc_layer_ro(aggp2, degp3, W1_1, b1_1, W2_1, b2_1, W_ro, b_ro)


# speedup vs baseline: 1.1431x; 1.1431x over previous
import functools

import jax
import jax.numpy as jnp
from jax import lax
from jax.experimental import pallas as pl
from jax.experimental.pallas import tpu as pltpu
from jax.experimental.pallas import tpu_sc as plsc

N = 10000
E = N * 32
H = 128

NC = 2
NS = 16
NW = NC * NS
EPW = E // NW
C = 125
CHUNKS = EPW // C
ROWS_PER_TILE = 624
TAIL_ROWS = N - NS * ROWS_PER_TILE

_NEG_SLOPE = 0.2


def _lrelu(x):
    return jnp.where(x >= 0, x, _NEG_SLOPE * x)


def _make_sc_agg(compute_deg: bool):
    mesh = plsc.VectorSubcoreMesh(core_axis_name="c", subcore_axis_name="s")
    if compute_deg:
        out_type = [jax.ShapeDtypeStruct((NC, N, H), jnp.float32),
                    jax.ShapeDtypeStruct((NC, N), jnp.float32)]
    else:
        out_type = jax.ShapeDtypeStruct((NC, N, H), jnp.float32)
    scratch_types = (
        [pltpu.VMEM((2, C), jnp.int32) for _ in range(4)]
        + [pltpu.VMEM((C, H), jnp.float32) for _ in range(2)]
        + [
            pltpu.VMEM((128,), jnp.float32),
            pltpu.VMEM_SHARED((N, H), jnp.float32),
            pltpu.VMEM_SHARED((N,), jnp.float32),
        ]
        + [pltpu.SemaphoreType.DMA for _ in range(6)]
    )

    def body(h_hbm, idx_hbm, zrows_hbm, zdeg_hbm, *refs):
        if compute_deg:
            agg_out, deg_out = refs[0], refs[1]
            rest = refs[2:]
        else:
            agg_out = refs[0]
            deg_out = None
            rest = refs[1:]
        idxb = rest[:4]
        rowsb = rest[4:6]
        ones_v, agg_sh, deg_sh = rest[6:9]
        gsem = rest[9:11]
        isem = rest[11:15]

        c = lax.axis_index("c")
        s = lax.axis_index("s")
        wid = c * NS + s

        pltpu.sync_copy(zrows_hbm,
                        agg_sh.at[pl.ds(s * ROWS_PER_TILE, ROWS_PER_TILE)])

        @pl.when(s == NS - 1)
        def _():
            pltpu.sync_copy(zrows_hbm.at[pl.ds(0, TAIL_ROWS)],
                            agg_sh.at[pl.ds(NS * ROWS_PER_TILE, TAIL_ROWS)])

        if compute_deg:
            @pl.when(s == 0)
            def _():
                pltpu.sync_copy(zdeg_hbm, deg_sh)
            one16 = jnp.ones((16,), jnp.float32)
            for j in range(8):
                ones_v[pl.ds(j * 16, 16)] = one16
        plsc.subcore_barrier()

        pltpu.sync_copy(idx_hbm.at[wid, 0], idxb[0])
        pltpu.sync_copy(idx_hbm.at[wid, 1], idxb[1])
        pltpu.async_copy(h_hbm.at[idxb[0].at[0]], rowsb[0], gsem[0])
        pltpu.async_copy(h_hbm.at[idxb[1].at[0]], rowsb[1], gsem[1])
        pltpu.async_copy(idx_hbm.at[wid, 2], idxb[2], isem[2])
        pltpu.async_copy(idx_hbm.at[wid, 3], idxb[3], isem[3])

        def round_body(j, carry):
            q0 = j * 4
            for t in range(4):
                q = q0 + t
                r = t % 2
                pltpu.make_async_copy(h_hbm.at[idxb[t].at[0]],
                                      rowsb[r], gsem[r]).wait()
                pltpu.sync_copy(rowsb[r], agg_sh.at[idxb[t].at[1]], add=True)
                if compute_deg:
                    pltpu.sync_copy(ones_v.at[pl.ds(0, C)],
                                    deg_sh.at[idxb[t].at[1]], add=True)

                tn = (t + 2) % 4
                @pl.when(q + 2 < CHUNKS)
                def _(t=t, tn=tn, r=r, q=q):
                    pltpu.make_async_copy(idx_hbm.at[wid, q + 2],
                                          idxb[tn], isem[tn]).wait()
                    pltpu.async_copy(h_hbm.at[idxb[tn].at[0]], rowsb[r], gsem[r])

                @pl.when(q + 4 < CHUNKS)
                def _(t=t, q=q):
                    pltpu.async_copy(idx_hbm.at[wid, q + 4], idxb[t], isem[t])
            return carry

        lax.fori_loop(0, CHUNKS // 4, round_body, 0)
        plsc.subcore_barrier()

        r0 = s * ROWS_PER_TILE
        pltpu.sync_copy(agg_sh.at[pl.ds(r0, ROWS_PER_TILE)],
                        agg_out.at[c, pl.ds(r0, ROWS_PER_TILE)])

        @pl.when(s == NS - 1)
        def _():
            rt = NS * ROWS_PER_TILE
            pltpu.sync_copy(agg_sh.at[pl.ds(rt, TAIL_ROWS)],
                            agg_out.at[c, pl.ds(rt, TAIL_ROWS)])

        if compute_deg:
            @pl.when(s == 0)
            def _():
                pltpu.sync_copy(deg_sh, deg_out.at[c])

    return functools.partial(pl.kernel, mesh=mesh, out_type=out_type,
                             scratch_types=scratch_types)(body)


_sc_agg_deg = _make_sc_agg(True)
_sc_agg = _make_sc_agg(False)


BLK = 1000


def _embed_body(x_ref, w_ref, b_ref, o_ref):
    o_ref[...] = jnp.dot(x_ref[...], w_ref[...],
                         preferred_element_type=jnp.float32) + b_ref[...]


def _tc_embed(x, w, b):
    d = x.shape[1]
    return pl.pallas_call(
        _embed_body,
        grid=(N // BLK,),
        in_specs=[
            pl.BlockSpec((BLK, d), lambda i: (i, 0)),
            pl.BlockSpec((d, H), lambda i: (0, 0)),
            pl.BlockSpec((1, H), lambda i: (0, 0)),
        ],
        out_specs=pl.BlockSpec((BLK, H), lambda i: (i, 0)),
        out_shape=jax.ShapeDtypeStruct((N, H), jnp.float32),
    )(x, w, b.reshape(1, H))


def _layer_body(p_ref, deg_ref, w1_ref, b1_ref, w2_ref, b2_ref, o_ref):
    agg = p_ref[0] + p_ref[1]
    deg = deg_ref[0] + deg_ref[1]
    agg = agg / jnp.maximum(deg, 1.0)
    t = _lrelu(jnp.dot(agg, w1_ref[...],
                       preferred_element_type=jnp.float32) + b1_ref[...])
    o_ref[...] = _lrelu(jnp.dot(t, w2_ref[...],
                                preferred_element_type=jnp.float32) + b2_ref[...])


def _tc_layer(partials, degp, w1, b1, w2, b2):
    return pl.pallas_call(
        _layer_body,
        grid=(N // BLK,),
        in_specs=[
            pl.BlockSpec((NC, BLK, H), lambda i: (0, i, 0)),
            pl.BlockSpec((NC, BLK, 1), lambda i: (0, i, 0)),
            pl.BlockSpec((H, H), lambda i: (0, 0)),
            pl.BlockSpec((1, H), lambda i: (0, 0)),
            pl.BlockSpec((H, H), lambda i: (0, 0)),
            pl.BlockSpec((1, H), lambda i: (0, 0)),
        ],
        out_specs=pl.BlockSpec((BLK, H), lambda i: (i, 0)),
        out_shape=jax.ShapeDtypeStruct((N, H), jnp.float32),
    )(partials, degp, w1, b1.reshape(1, H), w2, b2.reshape(1, H))


def _layer_ro_body(p_ref, deg_ref, w1_ref, b1_ref, w2_ref, b2_ref,
                   wro_ref, bro_ref, o_ref):
    agg = p_ref[0] + p_ref[1]
    deg = deg_ref[0] + deg_ref[1]
    agg = agg / jnp.maximum(deg, 1.0)
    t = _lrelu(jnp.dot(agg, w1_ref[...],
                       preferred_element_type=jnp.float32) + b1_ref[...])
    h = _lrelu(jnp.dot(t, w2_ref[...],
                       preferred_element_type=jnp.float32) + b2_ref[...])
    o_ref[...] = jnp.dot(h, wro_ref[...],
                         preferred_element_type=jnp.float32) + bro_ref[...]


def _tc_layer_ro(partials, degp, w1, b1, w2, b2, wro, bro):
    o = wro.shape[1]
    return pl.pallas_call(
        _layer_ro_body,
        grid=(N // BLK,),
        in_specs=[
            pl.BlockSpec((NC, BLK, H), lambda i: (0, i, 0)),
            pl.BlockSpec((NC, BLK, 1), lambda i: (0, i, 0)),
            pl.BlockSpec((H, H), lambda i: (0, 0)),
            pl.BlockSpec((1, H), lambda i: (0, 0)),
            pl.BlockSpec((H, H), lambda i: (0, 0)),
            pl.BlockSpec((1, H), lambda i: (0, 0)),
            pl.BlockSpec((H, o), lambda i: (0, 0)),
            pl.BlockSpec((1, o), lambda i: (0, 0)),
        ],
        out_specs=pl.BlockSpec((BLK, o), lambda i: (i, 0)),
        out_shape=jax.ShapeDtypeStruct((N, o), jnp.float32),
    )(partials, degp, w1, b1.reshape(1, H), w2, b2.reshape(1, H),
      wro, bro.reshape(1, o))


def kernel(feats, edge_index, W_emb, b_emb, W1_0, b1_0, W2_0, b2_0,
           W1_1, b1_1, W2_1, b2_1, W_ro, b_ro):
    idx = jnp.stack([edge_index[0].reshape(NW, CHUNKS, C),
                     edge_index[1].reshape(NW, CHUNKS, C)], axis=2)
    zrows = jnp.zeros((ROWS_PER_TILE, H), jnp.float32)
    zdeg = jnp.zeros((N,), jnp.float32)

    h0 = _tc_embed(feats, W_emb, b_emb)
    aggp, degp = _sc_agg_deg(h0, idx, zrows, zdeg)
    degp3 = degp.reshape(NC, N, 1)
    h1 = _tc_layer(aggp, degp3, W1_0, b1_0, W2_0, b2_0)
    aggp2 = _sc_agg(h1, idx, zrows, zdeg)
    return _tc_layer_ro(aggp2, degp3, W1_1, b1_1, W2_1, b2_1, W_ro, b_ro)


# speedup vs baseline: 1.1774x; 1.0300x over previous
import functools

import jax
import jax.numpy as jnp
from jax import lax
from jax.experimental import pallas as pl
from jax.experimental.pallas import tpu as pltpu
from jax.experimental.pallas import tpu_sc as plsc

N = 10000
E = N * 32
H = 128

NC = 2
NS = 16
NW = NC * NS
EPW = E // NW
C = 125
CHUNKS = EPW // C
ROWS_PER_TILE = 624
TAIL_ROWS = N - NS * ROWS_PER_TILE

_NEG_SLOPE = 0.2


def _lrelu(x):
    return jnp.where(x >= 0, x, _NEG_SLOPE * x)


def _make_sc_agg(compute_deg: bool):
    mesh = plsc.VectorSubcoreMesh(core_axis_name="c", subcore_axis_name="s")
    if compute_deg:
        out_type = [jax.ShapeDtypeStruct((NC, N, H), jnp.float32),
                    jax.ShapeDtypeStruct((NC, N), jnp.float32)]
    else:
        out_type = jax.ShapeDtypeStruct((NC, N, H), jnp.float32)
    scratch_types = (
        [pltpu.VMEM((2, C), jnp.int32) for _ in range(4)]
        + [pltpu.VMEM((C, H), jnp.float32) for _ in range(2)]
        + [
            pltpu.VMEM((128,), jnp.float32),
            pltpu.VMEM_SHARED((N, H), jnp.float32),
            pltpu.VMEM_SHARED((N,), jnp.float32),
        ]
        + [pltpu.SemaphoreType.DMA for _ in range(6)]
    )

    def body(h_hbm, idx_hbm, zrows_hbm, zdeg_hbm, *refs):
        if compute_deg:
            agg_out, deg_out = refs[0], refs[1]
            rest = refs[2:]
        else:
            agg_out = refs[0]
            deg_out = None
            rest = refs[1:]
        idxb = rest[:4]
        rowsb = rest[4:6]
        ones_v, agg_sh, deg_sh = rest[6:9]
        gsem = rest[9:11]
        isem = rest[11:15]

        c = lax.axis_index("c")
        s = lax.axis_index("s")
        wid = c * NS + s

        pltpu.sync_copy(zrows_hbm,
                        agg_sh.at[pl.ds(s * ROWS_PER_TILE, ROWS_PER_TILE)])

        @pl.when(s == NS - 1)
        def _():
            pltpu.sync_copy(zrows_hbm.at[pl.ds(0, TAIL_ROWS)],
                            agg_sh.at[pl.ds(NS * ROWS_PER_TILE, TAIL_ROWS)])

        if compute_deg:
            @pl.when(s == 0)
            def _():
                pltpu.sync_copy(zdeg_hbm, deg_sh)
            one16 = jnp.ones((16,), jnp.float32)
            for j in range(8):
                ones_v[pl.ds(j * 16, 16)] = one16
        plsc.subcore_barrier()

        pltpu.sync_copy(idx_hbm.at[wid, 0], idxb[0])
        pltpu.sync_copy(idx_hbm.at[wid, 1], idxb[1])
        pltpu.async_copy(h_hbm.at[idxb[0].at[0]], rowsb[0], gsem[0])
        pltpu.async_copy(h_hbm.at[idxb[1].at[0]], rowsb[1], gsem[1])
        pltpu.async_copy(idx_hbm.at[wid, 2], idxb[2], isem[2])
        pltpu.async_copy(idx_hbm.at[wid, 3], idxb[3], isem[3])

        def round_body(j, carry):
            q0 = j * 4
            for t in range(4):
                q = q0 + t
                r = t % 2
                pltpu.make_async_copy(h_hbm.at[idxb[t].at[0]],
                                      rowsb[r], gsem[r]).wait()
                pltpu.sync_copy(rowsb[r], agg_sh.at[idxb[t].at[1]], add=True)
                if compute_deg:
                    pltpu.sync_copy(ones_v.at[pl.ds(0, C)],
                                    deg_sh.at[idxb[t].at[1]], add=True)

                tn = (t + 2) % 4
                @pl.when(q + 2 < CHUNKS)
                def _(t=t, tn=tn, r=r, q=q):
                    pltpu.make_async_copy(idx_hbm.at[wid, q + 2],
                                          idxb[tn], isem[tn]).wait()
                    pltpu.async_copy(h_hbm.at[idxb[tn].at[0]], rowsb[r], gsem[r])

                @pl.when(q + 4 < CHUNKS)
                def _(t=t, q=q):
                    pltpu.async_copy(idx_hbm.at[wid, q + 4], idxb[t], isem[t])
            return carry

        lax.fori_loop(0, CHUNKS // 4, round_body, 0)
        plsc.subcore_barrier()

        r0 = s * ROWS_PER_TILE
        pltpu.sync_copy(agg_sh.at[pl.ds(r0, ROWS_PER_TILE)],
                        agg_out.at[c, pl.ds(r0, ROWS_PER_TILE)])

        @pl.when(s == NS - 1)
        def _():
            rt = NS * ROWS_PER_TILE
            pltpu.sync_copy(agg_sh.at[pl.ds(rt, TAIL_ROWS)],
                            agg_out.at[c, pl.ds(rt, TAIL_ROWS)])

        if compute_deg:
            @pl.when(s == 0)
            def _():
                pltpu.sync_copy(deg_sh, deg_out.at[c])

    return functools.partial(pl.kernel, mesh=mesh, out_type=out_type,
                             scratch_types=scratch_types)(body)


_sc_agg_deg = _make_sc_agg(True)
_sc_agg = _make_sc_agg(False)


BLK = 1000


def _layer1_body(p_ref, deg_ref, we_ref, be_ref, w1_ref, b1_ref,
                 w2_ref, b2_ref, o_ref):
    deg = deg_ref[0] + deg_ref[1]
    aggf = (p_ref[0] + p_ref[1]) / jnp.maximum(deg, 1.0)
    agg = jnp.dot(aggf, we_ref[...],
                  preferred_element_type=jnp.float32) + be_ref[...]
    t = _lrelu(jnp.dot(agg, w1_ref[...],
                       preferred_element_type=jnp.float32) + b1_ref[...])
    o_ref[...] = _lrelu(jnp.dot(t, w2_ref[...],
                                preferred_element_type=jnp.float32) + b2_ref[...])


def _tc_layer1(partials, degp, we, be, w1, b1, w2, b2):
    d = we.shape[0]
    return pl.pallas_call(
        _layer1_body,
        grid=(N // BLK,),
        in_specs=[
            pl.BlockSpec((NC, BLK, d), lambda i: (0, i, 0)),
            pl.BlockSpec((NC, BLK, 1), lambda i: (0, i, 0)),
            pl.BlockSpec((d, H), lambda i: (0, 0)),
            pl.BlockSpec((1, H), lambda i: (0, 0)),
            pl.BlockSpec((H, H), lambda i: (0, 0)),
            pl.BlockSpec((1, H), lambda i: (0, 0)),
            pl.BlockSpec((H, H), lambda i: (0, 0)),
            pl.BlockSpec((1, H), lambda i: (0, 0)),
        ],
        out_specs=pl.BlockSpec((BLK, H), lambda i: (i, 0)),
        out_shape=jax.ShapeDtypeStruct((N, H), jnp.float32),
    )(partials, degp, we, be.reshape(1, H), w1, b1.reshape(1, H),
      w2, b2.reshape(1, H))


def _layer_ro_body(p_ref, deg_ref, w1_ref, b1_ref, w2_ref, b2_ref,
                   wro_ref, bro_ref, o_ref):
    agg = p_ref[0] + p_ref[1]
    deg = deg_ref[0] + deg_ref[1]
    agg = agg / jnp.maximum(deg, 1.0)
    t = _lrelu(jnp.dot(agg, w1_ref[...],
                       preferred_element_type=jnp.float32) + b1_ref[...])
    h = _lrelu(jnp.dot(t, w2_ref[...],
                       preferred_element_type=jnp.float32) + b2_ref[...])
    o_ref[...] = jnp.dot(h, wro_ref[...],
                         preferred_element_type=jnp.float32) + bro_ref[...]


def _tc_layer_ro(partials, degp, w1, b1, w2, b2, wro, bro):
    o = wro.shape[1]
    return pl.pallas_call(
        _layer_ro_body,
        grid=(N // BLK,),
        in_specs=[
            pl.BlockSpec((NC, BLK, H), lambda i: (0, i, 0)),
            pl.BlockSpec((NC, BLK, 1), lambda i: (0, i, 0)),
            pl.BlockSpec((H, H), lambda i: (0, 0)),
            pl.BlockSpec((1, H), lambda i: (0, 0)),
            pl.BlockSpec((H, H), lambda i: (0, 0)),
            pl.BlockSpec((1, H), lambda i: (0, 0)),
            pl.BlockSpec((H, o), lambda i: (0, 0)),
            pl.BlockSpec((1, o), lambda i: (0, 0)),
        ],
        out_specs=pl.BlockSpec((BLK, o), lambda i: (i, 0)),
        out_shape=jax.ShapeDtypeStruct((N, o), jnp.float32),
    )(partials, degp, w1, b1.reshape(1, H), w2, b2.reshape(1, H),
      wro, bro.reshape(1, o))


def kernel(feats, edge_index, W_emb, b_emb, W1_0, b1_0, W2_0, b2_0,
           W1_1, b1_1, W2_1, b2_1, W_ro, b_ro):
    idx = jnp.stack([edge_index[0].reshape(NW, CHUNKS, C),
                     edge_index[1].reshape(NW, CHUNKS, C)], axis=2)
    zrows = jnp.zeros((ROWS_PER_TILE, H), jnp.float32)
    zdeg = jnp.zeros((N,), jnp.float32)

    aggp, degp = _sc_agg_deg(feats, idx, zrows, zdeg)
    degp3 = degp.reshape(NC, N, 1)
    h1 = _tc_layer1(aggp, degp3, W_emb, b_emb, W1_0, b1_0, W2_0, b2_0)
    aggp2 = _sc_agg(h1, idx, zrows, zdeg)
    return _tc_layer_ro(aggp2, degp3, W1_1, b1_1, W2_1, b2_1, W_ro, b_ro)
